# baseline (speedup 1.0000x reference)
import functools

import jax
import jax.numpy as jnp
from jax import lax
from jax.experimental import pallas as pl
from jax.experimental.pallas import tpu as pltpu
from jax.experimental.pallas import tpu_sc as plsc

NUM_CLASS = 100000
BATCH = 1024
CTX_DIM = 512
N_CLS_CTX = 4
PREFIX_LEN = 5
SUFFIX_LEN = 68
CLIP_LEN = 77

ROW = CLIP_LEN * CTX_DIM
PRE = PREFIX_LEN * CTX_DIM
MID = N_CLS_CTX * CTX_DIM
SUF = SUFFIX_LEN * CTX_DIM

NC, NS = 2, 16
NW = NC * NS
BPW = BATCH // NW

_mesh = plsc.VectorSubcoreMesh(core_axis_name="c", subcore_axis_name="s")


@functools.partial(
    pl.kernel,
    mesh=_mesh,
    out_type=jax.ShapeDtypeStruct((BATCH, ROW), jnp.float32),
    scratch_types=[
        pltpu.VMEM((BPW,), jnp.int32),
        pltpu.VMEM((BPW, MID), jnp.float32),
        pltpu.VMEM((PRE,), jnp.float32),
        pltpu.VMEM((SUF,), jnp.float32),
        pltpu.SemaphoreType.DMA,
        pltpu.SemaphoreType.DMA,
    ],
)
def _prompt_kernel(label_hbm, table_hbm, prefix_hbm, suffix_hbm, out_hbm,
                   idx_v, rows_v, pre_v, suf_v, gsem, wsem):
    wid = lax.axis_index("s") * NC + lax.axis_index("c")
    base = wid * BPW
    pltpu.sync_copy(label_hbm.at[pl.ds(base, BPW)], idx_v)
    pltpu.sync_copy(prefix_hbm, pre_v)
    pltpu.sync_copy(suffix_hbm, suf_v)
    pltpu.async_copy(table_hbm.at[idx_v], rows_v, gsem).wait()
    cps = []
    for j in range(BPW):
        b = base + j
        cps.append(pltpu.async_copy(pre_v, out_hbm.at[b, pl.ds(0, PRE)], wsem))
        cps.append(pltpu.async_copy(rows_v.at[j], out_hbm.at[b, pl.ds(PRE, MID)], wsem))
        cps.append(pltpu.async_copy(suf_v, out_hbm.at[b, pl.ds(PRE + MID, SUF)], wsem))
    for cp in cps:
        cp.wait()


def kernel(label, cls_ctx, token_prefix, token_suffix):
    table = cls_ctx.reshape(NUM_CLASS, MID)
    pre = token_prefix.reshape(PRE)
    suf = token_suffix.reshape(SUF)
    out = _prompt_kernel(label.astype(jnp.int32), table, pre, suf)
    return out.reshape(BATCH, CLIP_LEN, CTX_DIM)
